# Initial kernel scaffold; baseline (speedup 1.0000x reference)
import jax
import jax.numpy as jnp
from jax.experimental import pallas as pl


def kernel(x, edge_index, W_l, b_l, W_r, W_lin, b_lin):
    raise NotImplementedError("write your pallas kernel here")


# speedup vs baseline: 82.7351x; 82.7351x over previous
import functools

import jax
import jax.numpy as jnp
from jax import lax
from jax.experimental import pallas as pl
from jax.experimental.pallas import tpu as pltpu
from jax.experimental.pallas import tpu_sc as plsc

N = 100000
E = 3200000
NPAD = 102400
RP = NPAD // 16
PW = 8
ROWS = 10
LANES = 128
CHUNK = ROWS * LANES
NCHUNKS = E // CHUNK
NW = 32
ITERS = (NCHUNKS + NW - 1) // NW

_f32 = jnp.float32
_i32 = jnp.int32


def _sc_aggregate(x_pair, zeros, edge4):
    mesh = plsc.VectorSubcoreMesh(core_axis_name="c", subcore_axis_name="s")

    @functools.partial(
        pl.kernel,
        out_type=jax.ShapeDtypeStruct((2, NPAD, PW), _f32),
        mesh=mesh,
        scratch_types=(
            [pltpu.VMEM((LANES,), _i32) for _ in range(ROWS)]
            + [pltpu.VMEM((LANES,), _i32) for _ in range(ROWS)]
            + [pltpu.VMEM((LANES, PW), _f32) for _ in range(ROWS)]
            + [
                pltpu.VMEM_SHARED((NPAD, PW), _f32),
                pltpu.VMEM_SHARED((NPAD, PW), _f32),
                pltpu.SemaphoreType.DMA,
            ]
        ),
        compiler_params=pltpu.CompilerParams(use_tc_tiling_on_sc=False),
    )
    def sc_agg(xp_hbm, z_hbm, edge_hbm, out_hbm, *refs):
        src_v = refs[0:ROWS]
        dst_v = refs[ROWS:2 * ROWS]
        val_v = refs[2 * ROWS:3 * ROWS]
        x_sp, acc_sp, sem = refs[3 * ROWS:]
        cid = lax.axis_index("c")
        sid = lax.axis_index("s")
        wid = sid * 2 + cid
        off = sid * RP

        pltpu.sync_copy(xp_hbm.at[pl.ds(off, RP)], x_sp.at[pl.ds(off, RP)])
        pltpu.sync_copy(z_hbm.at[pl.ds(off, RP)], acc_sp.at[pl.ds(off, RP)])

        plsc.subcore_barrier()

        def chunk_body(i, _):
            t = wid + NW * i

            @pl.when(t < NCHUNKS)
            def _():
                idx = [
                    pltpu.async_copy(edge_hbm.at[0, t, r], src_v[r], sem)
                    for r in range(ROWS)
                ] + [
                    pltpu.async_copy(edge_hbm.at[1, t, r], dst_v[r], sem)
                    for r in range(ROWS)
                ]
                for c in idx:
                    c.wait()
                gets = [
                    pltpu.async_copy(x_sp.at[src_v[r]], val_v[r], sem)
                    for r in range(ROWS)
                ]
                for c in gets:
                    c.wait()
                puts = [
                    pltpu.async_copy(val_v[r], acc_sp.at[dst_v[r]],
                                     sem, add=True)
                    for r in range(ROWS)
                ]
                for c in puts:
                    c.wait()

            return 0

        lax.fori_loop(0, ITERS, chunk_body, 0)

        plsc.subcore_barrier()

        pltpu.sync_copy(acc_sp.at[pl.ds(off, RP)],
                        out_hbm.at[cid, pl.ds(off, RP)])

    return sc_agg(x_pair, zeros, edge4)


def _epilogue(a0, a1, c0, c1, x2, wl, bl, wr, wlin, blin):
    def body(a0_r, a1_r, c0_r, c1_r, x_r, wl_r, bl_r, wr_r, wlin_r, blin_r, out_r):
        agg = a0_r[...] + a1_r[...]
        cnt = jnp.maximum(c0_r[...] + c1_r[...], 1.0)
        mean = agg / cnt
        xv = x_r[...]

        kk = lax.broadcasted_iota(_i32, (100, 400), 0)
        jj = lax.broadcasted_iota(_i32, (100, 400), 1)
        f = jj - 4 * (jj // 4)
        sel = (jj // 4) == kk

        def expand(w_r):
            v = jnp.where(
                f == 0, w_r[0, 0],
                jnp.where(f == 1, w_r[0, 1],
                          jnp.where(f == 2, w_r[0, 2], w_r[0, 3])))
            return jnp.where(sel, v, 0.0)

        s_l = expand(wl_r)
        s_r = expand(wr_r)

        j2 = lax.broadcasted_iota(_i32, (8, 400), 1)
        f2 = j2 - 4 * (j2 // 4)
        brow = jnp.where(
            f2 == 0, bl_r[0, 0],
            jnp.where(f2 == 1, bl_r[0, 1],
                      jnp.where(f2 == 2, bl_r[0, 2], bl_r[0, 3])))[:1]

        h = (jax.lax.dot(mean, s_l, precision=jax.lax.Precision.HIGHEST,
                         preferred_element_type=_f32)
             + jax.lax.dot(xv, s_r, precision=jax.lax.Precision.HIGHEST,
                           preferred_element_type=_f32))
        h = jnp.maximum(h + brow, 0.0)
        out = jax.lax.dot_general(
            h, wlin_r[...], (((1,), (1,)), ((), ())),
            precision=jax.lax.Precision.HIGHEST, preferred_element_type=_f32)
        out_r[...] = out + blin_r[...]

    return pl.pallas_call(
        body,
        out_shape=jax.ShapeDtypeStruct((1000, 10), _f32),
    )(a0, a1, c0, c1, x2, wl, bl, wr, wlin, blin)


def kernel(x, edge_index, W_l, b_l, W_r, W_lin, b_lin):
    x_pair = jnp.pad(
        jnp.concatenate([x, jnp.ones_like(x)], axis=1),
        ((0, NPAD - N), (0, PW - 2)))
    zeros = jnp.zeros((NPAD, PW), _f32)
    edge4 = edge_index.reshape(2, NCHUNKS, ROWS, LANES)
    accP = _sc_aggregate(x_pair, zeros, edge4)

    a0 = accP[0, :N, 0].reshape(1000, 100)
    a1 = accP[1, :N, 0].reshape(1000, 100)
    c0 = accP[0, :N, 1].reshape(1000, 100)
    c1 = accP[1, :N, 1].reshape(1000, 100)
    x2 = x.reshape(1000, 100)
    wl = W_l.reshape(1, 4)
    wr = W_r.reshape(1, 4)
    bl = b_l.reshape(1, 4)
    blin = b_lin.reshape(1, 10)
    return _epilogue(a0, a1, c0, c1, x2, wl, bl, wr, W_lin, blin)


# speedup vs baseline: 85.8839x; 1.0381x over previous
import functools

import jax
import jax.numpy as jnp
from jax import lax
from jax.experimental import pallas as pl
from jax.experimental.pallas import tpu as pltpu
from jax.experimental.pallas import tpu_sc as plsc

N = 100000
E = 3200000
NPAD = 102400
RP = NPAD // 16
PW = 8
ROWS = 10
LANES = 128
CHUNK = ROWS * LANES
NCHUNKS = E // CHUNK
NW = 32
ITERS = (NCHUNKS + NW - 1) // NW

_f32 = jnp.float32
_i32 = jnp.int32


def _sc_aggregate(x_pair, zeros, edge4):
    mesh = plsc.VectorSubcoreMesh(core_axis_name="c", subcore_axis_name="s")

    @functools.partial(
        pl.kernel,
        out_type=jax.ShapeDtypeStruct((2, NPAD, PW), _f32),
        mesh=mesh,
        scratch_types=(
            [pltpu.VMEM((LANES,), _i32) for _ in range(ROWS)]
            + [pltpu.VMEM((LANES,), _i32) for _ in range(ROWS)]
            + [pltpu.VMEM((LANES, PW), _f32) for _ in range(ROWS)]
            + [
                pltpu.VMEM_SHARED((NPAD, PW), _f32),
                pltpu.VMEM_SHARED((NPAD, PW), _f32),
                pltpu.SemaphoreType.DMA,
            ]
        ),
        compiler_params=pltpu.CompilerParams(use_tc_tiling_on_sc=False),
    )
    def sc_agg(xp_hbm, z_hbm, edge_hbm, out_hbm, *refs):
        src_v = refs[0:ROWS]
        dst_v = refs[ROWS:2 * ROWS]
        val_v = refs[2 * ROWS:3 * ROWS]
        x_sp, acc_sp, sem = refs[3 * ROWS:]
        cid = lax.axis_index("c")
        sid = lax.axis_index("s")
        wid = sid * 2 + cid
        off = sid * RP

        pltpu.sync_copy(xp_hbm.at[pl.ds(off, RP)], x_sp.at[pl.ds(off, RP)])
        pltpu.sync_copy(z_hbm.at[pl.ds(off, RP)], acc_sp.at[pl.ds(off, RP)])

        plsc.subcore_barrier()

        def chunk_body(i, _):
            t = wid + NW * i

            @pl.when(t < NCHUNKS)
            def _():
                idx = [
                    pltpu.async_copy(edge_hbm.at[0, t, r], src_v[r], sem)
                    for r in range(ROWS)
                ] + [
                    pltpu.async_copy(edge_hbm.at[1, t, r], dst_v[r], sem)
                    for r in range(ROWS)
                ]
                for c in idx:
                    c.wait()
                gets = [
                    pltpu.async_copy(x_sp.at[src_v[r]], val_v[r], sem)
                    for r in range(ROWS)
                ]
                puts = []
                for r in range(ROWS):
                    gets[r].wait()
                    puts.append(
                        pltpu.async_copy(val_v[r], acc_sp.at[dst_v[r]],
                                         sem, add=True))
                for c in puts:
                    c.wait()

            return 0

        lax.fori_loop(0, ITERS, chunk_body, 0)

        plsc.subcore_barrier()

        pltpu.sync_copy(acc_sp.at[pl.ds(off, RP)],
                        out_hbm.at[cid, pl.ds(off, RP)])

    return sc_agg(x_pair, zeros, edge4)


def _epilogue(a0, a1, c0, c1, x2, wl, bl, wr, wlin, blin):
    def body(a0_r, a1_r, c0_r, c1_r, x_r, wl_r, bl_r, wr_r, wlin_r, blin_r, out_r):
        agg = a0_r[...] + a1_r[...]
        cnt = jnp.maximum(c0_r[...] + c1_r[...], 1.0)
        mean = agg / cnt
        xv = x_r[...]

        kk = lax.broadcasted_iota(_i32, (100, 400), 0)
        jj = lax.broadcasted_iota(_i32, (100, 400), 1)
        f = jj - 4 * (jj // 4)
        sel = (jj // 4) == kk

        def expand(w_r):
            v = jnp.where(
                f == 0, w_r[0, 0],
                jnp.where(f == 1, w_r[0, 1],
                          jnp.where(f == 2, w_r[0, 2], w_r[0, 3])))
            return jnp.where(sel, v, 0.0)

        s_l = expand(wl_r)
        s_r = expand(wr_r)

        j2 = lax.broadcasted_iota(_i32, (8, 400), 1)
        f2 = j2 - 4 * (j2 // 4)
        brow = jnp.where(
            f2 == 0, bl_r[0, 0],
            jnp.where(f2 == 1, bl_r[0, 1],
                      jnp.where(f2 == 2, bl_r[0, 2], bl_r[0, 3])))[:1]

        h = (jax.lax.dot(mean, s_l, precision=jax.lax.Precision.HIGHEST,
                         preferred_element_type=_f32)
             + jax.lax.dot(xv, s_r, precision=jax.lax.Precision.HIGHEST,
                           preferred_element_type=_f32))
        h = jnp.maximum(h + brow, 0.0)
        out = jax.lax.dot_general(
            h, wlin_r[...], (((1,), (1,)), ((), ())),
            precision=jax.lax.Precision.HIGHEST, preferred_element_type=_f32)
        out_r[...] = out + blin_r[...]

    return pl.pallas_call(
        body,
        out_shape=jax.ShapeDtypeStruct((1000, 10), _f32),
    )(a0, a1, c0, c1, x2, wl, bl, wr, wlin, blin)


def kernel(x, edge_index, W_l, b_l, W_r, W_lin, b_lin):
    x_pair = jnp.pad(
        jnp.concatenate([x, jnp.ones_like(x)], axis=1),
        ((0, NPAD - N), (0, PW - 2)))
    zeros = jnp.zeros((NPAD, PW), _f32)
    edge4 = edge_index.reshape(2, NCHUNKS, ROWS, LANES)
    accP = _sc_aggregate(x_pair, zeros, edge4)

    a0 = accP[0, :N, 0].reshape(1000, 100)
    a1 = accP[1, :N, 0].reshape(1000, 100)
    c0 = accP[0, :N, 1].reshape(1000, 100)
    c1 = accP[1, :N, 1].reshape(1000, 100)
    x2 = x.reshape(1000, 100)
    wl = W_l.reshape(1, 4)
    wr = W_r.reshape(1, 4)
    bl = b_l.reshape(1, 4)
    blin = b_lin.reshape(1, 10)
    return _epilogue(a0, a1, c0, c1, x2, wl, bl, wr, W_lin, blin)


# speedup vs baseline: 96.4392x; 1.1229x over previous
import functools

import jax
import jax.numpy as jnp
from jax import lax
from jax.experimental import pallas as pl
from jax.experimental.pallas import tpu as pltpu
from jax.experimental.pallas import tpu_sc as plsc

N = 100000
E = 3200000
NPAD = 102400
RP = NPAD // 16
PW = 8
ROWS = 10
LANES = 128
CHUNK = ROWS * LANES
NCHUNKS = E // CHUNK
NW = 32
ITERS = (NCHUNKS + NW - 1) // NW

_f32 = jnp.float32
_i32 = jnp.int32


def _sc_aggregate(x_pair, zeros, edge4):
    mesh = plsc.VectorSubcoreMesh(core_axis_name="c", subcore_axis_name="s")

    @functools.partial(
        pl.kernel,
        out_type=jax.ShapeDtypeStruct((2, NPAD, PW), _f32),
        mesh=mesh,
        scratch_types=(
            [pltpu.VMEM((LANES,), _i32) for _ in range(4 * ROWS)]
            + [pltpu.VMEM((LANES, PW), _f32) for _ in range(ROWS)]
            + [
                pltpu.VMEM_SHARED((NPAD, PW), _f32),
                pltpu.VMEM_SHARED((NPAD, PW), _f32),
                pltpu.SemaphoreType.DMA,
                pltpu.SemaphoreType.DMA,
            ]
        ),
        compiler_params=pltpu.CompilerParams(use_tc_tiling_on_sc=False),
    )
    def sc_agg(xp_hbm, z_hbm, edge_hbm, out_hbm, *refs):
        src_a = refs[0:ROWS]
        dst_a = refs[ROWS:2 * ROWS]
        src_b = refs[2 * ROWS:3 * ROWS]
        dst_b = refs[3 * ROWS:4 * ROWS]
        val_v = refs[4 * ROWS:5 * ROWS]
        x_sp, acc_sp, sem, isem = refs[5 * ROWS:]
        cid = lax.axis_index("c")
        sid = lax.axis_index("s")
        wid = sid * 2 + cid
        off = sid * RP

        pltpu.sync_copy(xp_hbm.at[pl.ds(off, RP)], x_sp.at[pl.ds(off, RP)])
        pltpu.sync_copy(z_hbm.at[pl.ds(off, RP)], acc_sp.at[pl.ds(off, RP)])

        plsc.subcore_barrier()

        def issue_idx(t, src_v, dst_v):
            for r in range(ROWS):
                pltpu.async_copy(edge_hbm.at[0, t, r], src_v[r], isem)
            for r in range(ROWS):
                pltpu.async_copy(edge_hbm.at[1, t, r], dst_v[r], isem)

        def wait_idx(t, src_v, dst_v):
            for r in range(ROWS):
                pltpu.make_async_copy(edge_hbm.at[0, t, r], src_v[r],
                                      isem).wait()
            for r in range(ROWS):
                pltpu.make_async_copy(edge_hbm.at[1, t, r], dst_v[r],
                                      isem).wait()

        def process(src_v, dst_v):
            gets = [
                pltpu.async_copy(x_sp.at[src_v[r]], val_v[r], sem)
                for r in range(ROWS)
            ]
            puts = []
            for r in range(ROWS):
                gets[r].wait()
                puts.append(
                    pltpu.async_copy(val_v[r], acc_sp.at[dst_v[r]],
                                     sem, add=True))
            for c in puts:
                c.wait()

        issue_idx(wid, src_a, dst_a)

        def chunk_body(i, _):
            ta = wid + NW * (2 * i)
            tb = ta + NW
            ta2 = ta + 2 * NW

            @pl.when(tb < NCHUNKS)
            def _():
                issue_idx(tb, src_b, dst_b)

            @pl.when(ta < NCHUNKS)
            def _():
                wait_idx(ta, src_a, dst_a)
                process(src_a, dst_a)

            @pl.when(ta2 < NCHUNKS)
            def _():
                issue_idx(ta2, src_a, dst_a)

            @pl.when(tb < NCHUNKS)
            def _():
                wait_idx(tb, src_b, dst_b)
                process(src_b, dst_b)

            return 0

        lax.fori_loop(0, (ITERS + 1) // 2, chunk_body, 0)

        plsc.subcore_barrier()

        pltpu.sync_copy(acc_sp.at[pl.ds(off, RP)],
                        out_hbm.at[cid, pl.ds(off, RP)])

    return sc_agg(x_pair, zeros, edge4)


def _epilogue(a0, a1, c0, c1, x2, wl, bl, wr, wlin, blin):
    def body(a0_r, a1_r, c0_r, c1_r, x_r, wl_r, bl_r, wr_r, wlin_r, blin_r, out_r):
        agg = a0_r[...] + a1_r[...]
        cnt = jnp.maximum(c0_r[...] + c1_r[...], 1.0)
        mean = agg / cnt
        xv = x_r[...]

        kk = lax.broadcasted_iota(_i32, (100, 400), 0)
        jj = lax.broadcasted_iota(_i32, (100, 400), 1)
        f = jj - 4 * (jj // 4)
        sel = (jj // 4) == kk

        def expand(w_r):
            v = jnp.where(
                f == 0, w_r[0, 0],
                jnp.where(f == 1, w_r[0, 1],
                          jnp.where(f == 2, w_r[0, 2], w_r[0, 3])))
            return jnp.where(sel, v, 0.0)

        s_l = expand(wl_r)
        s_r = expand(wr_r)

        j2 = lax.broadcasted_iota(_i32, (8, 400), 1)
        f2 = j2 - 4 * (j2 // 4)
        brow = jnp.where(
            f2 == 0, bl_r[0, 0],
            jnp.where(f2 == 1, bl_r[0, 1],
                      jnp.where(f2 == 2, bl_r[0, 2], bl_r[0, 3])))[:1]

        h = (jax.lax.dot(mean, s_l, precision=jax.lax.Precision.HIGHEST,
                         preferred_element_type=_f32)
             + jax.lax.dot(xv, s_r, precision=jax.lax.Precision.HIGHEST,
                           preferred_element_type=_f32))
        h = jnp.maximum(h + brow, 0.0)
        out = jax.lax.dot_general(
            h, wlin_r[...], (((1,), (1,)), ((), ())),
            precision=jax.lax.Precision.HIGHEST, preferred_element_type=_f32)
        out_r[...] = out + blin_r[...]

    return pl.pallas_call(
        body,
        out_shape=jax.ShapeDtypeStruct((1000, 10), _f32),
    )(a0, a1, c0, c1, x2, wl, bl, wr, wlin, blin)


def kernel(x, edge_index, W_l, b_l, W_r, W_lin, b_lin):
    x_pair = jnp.pad(
        jnp.concatenate([x, jnp.ones_like(x)], axis=1),
        ((0, NPAD - N), (0, PW - 2)))
    zeros = jnp.zeros((NPAD, PW), _f32)
    edge4 = edge_index.reshape(2, NCHUNKS, ROWS, LANES)
    accP = _sc_aggregate(x_pair, zeros, edge4)

    a0 = accP[0, :N, 0].reshape(1000, 100)
    a1 = accP[1, :N, 0].reshape(1000, 100)
    c0 = accP[0, :N, 1].reshape(1000, 100)
    c1 = accP[1, :N, 1].reshape(1000, 100)
    x2 = x.reshape(1000, 100)
    wl = W_l.reshape(1, 4)
    wr = W_r.reshape(1, 4)
    bl = b_l.reshape(1, 4)
    blin = b_lin.reshape(1, 10)
    return _epilogue(a0, a1, c0, c1, x2, wl, bl, wr, W_lin, blin)
